# baseline (speedup 1.0000x reference)
import functools

import jax
import jax.numpy as jnp
from jax import lax
from jax.experimental import pallas as pl
from jax.experimental.pallas import tpu as pltpu
from jax.experimental.pallas import tpu_sc as plsc

N = 16384
K = 1024
D = 64
BETA = 0.25
EPS = 1e-8

NH = N // 2
T = 2048
GH = NH // T

NW = 32
B_PER_W = NH // NW
DP = 128
NCH = B_PER_W // DP
WPS = T // B_PER_W


def _make_tc_body(fin):
    def body(x_ref, p_ref, soft_in, sse_in, idx_ref, soft_out, sse_out,
             *rest):
        if fin:
            loss_ref, ent_ref, sp_acc, iota_scr = rest
        else:
            sp_acc, iota_scr = rest
        g = pl.program_id(0)

        @pl.when(g == 0)
        def _init():
            p = p_ref[...]
            sp_acc[...] = jnp.sum(p * p, axis=1)[None, :]
            soft_out[...] = soft_in[...]
            sse_out[...] = sse_in[...]
            iota_scr[...] = lax.broadcasted_iota(jnp.int32, (1, K), 1).astype(
                jnp.float32)

        x = x_ref[...]
        sx = jnp.sum(x * x, axis=1, keepdims=True)
        mm = lax.dot_general(x, p_ref[...], (((1,), (1,)), ((), ())),
                             preferred_element_type=jnp.float32)
        d = sx + sp_acc[...] - 2.0 * mm
        dmin = jnp.min(d, axis=1, keepdims=True)
        idxf = jnp.min(jnp.where(d == dmin, iota_scr[...], float(K)),
                       axis=1)
        idx_ref[...] = idxf.astype(jnp.int32).reshape(WPS, NCH, DP)

        eb = jnp.exp(dmin - d).astype(jnp.bfloat16)
        onesk = jnp.ones((K, 1), jnp.bfloat16)
        rs = lax.dot_general(eb, onesk, (((1,), (0,)), ((), ())),
                             preferred_element_type=jnp.float32)
        recip = (1.0 / rs).astype(jnp.bfloat16)
        colsum = lax.dot_general(recip, eb, (((0,), (0,)), ((), ())),
                                 preferred_element_type=jnp.float32)
        soft_out[...] += colsum
        onest = jnp.ones((T, 1), jnp.bfloat16)
        sse_out[...] += lax.dot_general(dmin.astype(jnp.bfloat16), onest,
                                        (((0,), (0,)), ((), ())),
                                        preferred_element_type=jnp.float32)

        if fin:
            @pl.when(g == GH - 1)
            def _fini():
                s = soft_out[...] / N + EPS
                s = s / jnp.sum(s)
                ent_ref[...] = jnp.reshape(jnp.sum(-s * jnp.log(s)), (1, 1))
                loss_ref[...] = (1.0 + BETA) / (N * D) * sse_out[...]

    return body


def _make_tc_call(half, fin):
    n_out = 5 if fin else 3
    out_specs = [
        pl.BlockSpec((WPS, NCH, DP), lambda g: (g, 0, 0)),
        pl.BlockSpec((1, K), lambda g: (0, 0)),
        pl.BlockSpec((1, 1), lambda g: (0, 0)),
        pl.BlockSpec((1, 1), lambda g: (0, 0)),
        pl.BlockSpec((1, 1), lambda g: (0, 0)),
    ][:n_out]
    out_shape = [
        jax.ShapeDtypeStruct((NW, NCH, DP), jnp.int32),
        jax.ShapeDtypeStruct((1, K), jnp.float32),
        jax.ShapeDtypeStruct((1, 1), jnp.float32),
        jax.ShapeDtypeStruct((1, 1), jnp.float32),
        jax.ShapeDtypeStruct((1, 1), jnp.float32),
    ][:n_out]
    return pl.pallas_call(
        _make_tc_body(fin),
        grid=(GH,),
        in_specs=[
            pl.BlockSpec((T, D), lambda g: (g + half * GH, 0)),
            pl.BlockSpec((K, D), lambda g: (0, 0)),
            pl.BlockSpec((1, K), lambda g: (0, 0)),
            pl.BlockSpec((1, 1), lambda g: (0, 0)),
        ],
        out_specs=out_specs,
        out_shape=out_shape,
        scratch_shapes=[
            pltpu.VMEM((1, K), jnp.float32),
            pltpu.VMEM((1, K), jnp.float32),
        ],
    )


_tc_first = _make_tc_call(0, False)
_tc_final = _make_tc_call(1, True)


def _sc_gather_body(table_hbm, idx_hbm, out_hbm, idx_v, rows_v, sem):
    wid = lax.axis_index("s") * 2 + lax.axis_index("c")
    pltpu.sync_copy(idx_hbm.at[wid], idx_v)
    copies = [
        pltpu.async_copy(table_hbm.at[idx_v.at[j]],
                         rows_v.at[pl.ds(j * DP, DP)], sem)
        for j in range(NCH)
    ]
    for c in copies:
        c.wait()
    pltpu.sync_copy(rows_v, out_hbm.at[pl.ds(wid * B_PER_W, B_PER_W)])


@functools.cache
def _sc_gather():
    return functools.partial(
        pl.kernel,
        mesh=plsc.VectorSubcoreMesh(core_axis_name="c", subcore_axis_name="s"),
        out_type=jax.ShapeDtypeStruct((NH, DP), jnp.float32),
        scratch_types=[
            pltpu.VMEM((NCH, DP), jnp.int32),
            pltpu.VMEM((B_PER_W, DP), jnp.float32),
            pltpu.SemaphoreType.DMA,
        ],
    )(_sc_gather_body)


def kernel(latents, prototypes):
    z_soft = jnp.zeros((1, K), jnp.float32)
    z_sse = jnp.zeros((1, 1), jnp.float32)
    idx1, soft1, sse1 = _tc_first(latents, prototypes, z_soft, z_sse)
    idx2, _, _, loss, ent = _tc_final(latents, prototypes, soft1, sse1)
    table = jnp.pad(prototypes, ((0, 0), (0, DP - D)))
    o1 = _sc_gather()(table, idx1)
    o2 = _sc_gather()(table, idx2)
    quantized = jnp.concatenate([o1[:, :D], o2[:, :D]], axis=0)
    return quantized, loss[0, 0], ent[0, 0]


# speedup vs baseline: 1.2192x; 1.2192x over previous
import functools

import jax
import jax.numpy as jnp
from jax import lax
from jax.experimental import pallas as pl
from jax.experimental.pallas import tpu as pltpu
from jax.experimental.pallas import tpu_sc as plsc

N = 16384
K = 1024
D = 64
T = 2048
G = N // T
BETA = 0.25
EPS = 1e-8

NW = 32
B_PER_W = N // NW
DP = 128
NCH = B_PER_W // DP
WPS = T // B_PER_W


def _tc_body(xt_ref, pt_ref, idx_ref, loss_ref, ent_ref,
             soft_acc, sp_acc, sse_acc, iota_scr):
    g = pl.program_id(0)
    p = jnp.transpose(pt_ref[...])

    @pl.when(g == 0)
    def _init():
        sp_acc[...] = jnp.sum(p * p, axis=1)[None, :]
        soft_acc[...] = jnp.zeros_like(soft_acc)
        sse_acc[...] = jnp.zeros_like(sse_acc)
        iota_scr[...] = lax.broadcasted_iota(jnp.int32, (1, K), 1).astype(
            jnp.float32)

    x = jnp.transpose(xt_ref[...])
    sx = jnp.sum(x * x, axis=1, keepdims=True)
    mm = lax.dot_general(x, p, (((1,), (1,)), ((), ())),
                         preferred_element_type=jnp.float32)
    d = sx + sp_acc[...] - 2.0 * mm
    dmin = jnp.min(d, axis=1, keepdims=True)
    idxf = jnp.min(jnp.where(d == dmin, iota_scr[...], float(K)),
                   axis=1)
    idx_ref[...] = idxf.astype(jnp.int32).reshape(WPS, NCH, DP)

    eb = jnp.exp(dmin - d).astype(jnp.bfloat16)
    onesk = jnp.ones((K, 1), jnp.bfloat16)
    rs = lax.dot_general(eb, onesk, (((1,), (0,)), ((), ())),
                         preferred_element_type=jnp.float32)
    recip = (1.0 / rs).astype(jnp.bfloat16)
    colsum = lax.dot_general(recip, eb, (((0,), (0,)), ((), ())),
                             preferred_element_type=jnp.float32)
    soft_acc[...] += colsum
    onest = jnp.ones((T, 1), jnp.bfloat16)
    sse_acc[...] += lax.dot_general(dmin.astype(jnp.bfloat16), onest,
                                    (((0,), (0,)), ((), ())),
                                    preferred_element_type=jnp.float32)

    @pl.when(g == G - 1)
    def _fini():
        s = soft_acc[...] / N + EPS
        s = s / jnp.sum(s)
        ent_ref[...] = jnp.reshape(jnp.sum(-s * jnp.log(s)), (1, 1))
        loss_ref[...] = (1.0 + BETA) / (N * D) * sse_acc[...]


_tc_call = pl.pallas_call(
    _tc_body,
    grid=(G,),
    in_specs=[
        pl.BlockSpec((D, T), lambda g: (0, g)),
        pl.BlockSpec((D, K), lambda g: (0, 0)),
    ],
    out_specs=[
        pl.BlockSpec((WPS, NCH, DP), lambda g: (g, 0, 0)),
        pl.BlockSpec((1, 1), lambda g: (0, 0)),
        pl.BlockSpec((1, 1), lambda g: (0, 0)),
    ],
    out_shape=[
        jax.ShapeDtypeStruct((NW, NCH, DP), jnp.int32),
        jax.ShapeDtypeStruct((1, 1), jnp.float32),
        jax.ShapeDtypeStruct((1, 1), jnp.float32),
    ],
    scratch_shapes=[
        pltpu.VMEM((1, K), jnp.float32),
        pltpu.VMEM((1, K), jnp.float32),
        pltpu.VMEM((1, 1), jnp.float32),
        pltpu.VMEM((1, K), jnp.float32),
    ],
)


def _sc_gather_body(table_hbm, idx_hbm, out_hbm, idx_v, rows_v, sem):
    wid = lax.axis_index("s") * 2 + lax.axis_index("c")
    pltpu.sync_copy(idx_hbm.at[wid], idx_v)
    copies = [
        pltpu.async_copy(table_hbm.at[idx_v.at[j]],
                         rows_v.at[pl.ds(j * DP, DP)], sem)
        for j in range(NCH)
    ]
    for c in copies:
        c.wait()
    pltpu.sync_copy(rows_v, out_hbm.at[pl.ds(wid * B_PER_W, B_PER_W)])


@functools.cache
def _sc_gather():
    return functools.partial(
        pl.kernel,
        mesh=plsc.VectorSubcoreMesh(core_axis_name="c", subcore_axis_name="s"),
        out_type=jax.ShapeDtypeStruct((N, DP), jnp.float32),
        scratch_types=[
            pltpu.VMEM((NCH, DP), jnp.int32),
            pltpu.VMEM((B_PER_W, DP), jnp.float32),
            pltpu.SemaphoreType.DMA,
        ],
    )(_sc_gather_body)


def kernel(latents, prototypes):
    idx3, loss, ent = _tc_call(latents.T, prototypes.T)
    table = jnp.pad(prototypes, ((0, 0), (0, DP - D)))
    quantized = _sc_gather()(table, idx3)[:, :D]
    return quantized, loss[0, 0], ent[0, 0]


# speedup vs baseline: 1.2850x; 1.0540x over previous
import functools

import jax
import jax.numpy as jnp
from jax import lax
from jax.experimental import pallas as pl
from jax.experimental.pallas import tpu as pltpu
from jax.experimental.pallas import tpu_sc as plsc

N = 16384
K = 1024
D = 64
T = 4096
G = N // T
BETA = 0.25
EPS = 1e-8

NW = 32
B_PER_W = N // NW
DP = 128
NCH = B_PER_W // DP
WPS = T // B_PER_W


def _tc_body(xt_ref, pt_ref, idx_ref, loss_ref, ent_ref, table_ref,
             soft_acc, sp_acc, sse_acc, iota_scr):
    g = pl.program_id(0)
    p = jnp.transpose(pt_ref[...])

    @pl.when(g == 0)
    def _init():
        sp_acc[...] = jnp.sum(p * p, axis=1)[None, :]
        soft_acc[...] = jnp.zeros_like(soft_acc)
        sse_acc[...] = jnp.zeros_like(sse_acc)
        iota_scr[...] = lax.broadcasted_iota(jnp.int32, (1, K), 1).astype(
            jnp.float32)
        table_ref[...] = jnp.pad(p, ((0, 0), (0, DP - D)))

    x = jnp.transpose(xt_ref[...])
    sx = jnp.sum(x * x, axis=1, keepdims=True)
    mm = lax.dot_general(x, p, (((1,), (1,)), ((), ())),
                         preferred_element_type=jnp.float32)
    d = sx + sp_acc[...] - 2.0 * mm
    dmin = jnp.min(d, axis=1, keepdims=True)
    idxf = jnp.min(jnp.where(d == dmin, iota_scr[...], float(K)),
                   axis=1)
    idx_ref[...] = idxf.astype(jnp.int32).reshape(WPS, NCH, DP)

    eb = jnp.exp(dmin - d).astype(jnp.bfloat16)
    onesk = jnp.ones((K, 1), jnp.bfloat16)
    rs = lax.dot_general(eb, onesk, (((1,), (0,)), ((), ())),
                         preferred_element_type=jnp.float32)
    recip = (1.0 / rs).astype(jnp.bfloat16)
    colsum = lax.dot_general(recip, eb, (((0,), (0,)), ((), ())),
                             preferred_element_type=jnp.float32)
    soft_acc[...] += colsum
    onest = jnp.ones((T, 1), jnp.bfloat16)
    sse_acc[...] += lax.dot_general(dmin.astype(jnp.bfloat16), onest,
                                    (((0,), (0,)), ((), ())),
                                    preferred_element_type=jnp.float32)

    @pl.when(g == G - 1)
    def _fini():
        s = soft_acc[...] / N + EPS
        s = s / jnp.sum(s)
        ent_ref[...] = jnp.reshape(jnp.sum(-s * jnp.log(s)), (1, 1))
        loss_ref[...] = (1.0 + BETA) / (N * D) * sse_acc[...]


_tc_call = pl.pallas_call(
    _tc_body,
    grid=(G,),
    in_specs=[
        pl.BlockSpec((D, T), lambda g: (0, g)),
        pl.BlockSpec((D, K), lambda g: (0, 0)),
    ],
    out_specs=[
        pl.BlockSpec((WPS, NCH, DP), lambda g: (g, 0, 0)),
        pl.BlockSpec((1, 1), lambda g: (0, 0)),
        pl.BlockSpec((1, 1), lambda g: (0, 0)),
        pl.BlockSpec((K, DP), lambda g: (0, 0)),
    ],
    out_shape=[
        jax.ShapeDtypeStruct((NW, NCH, DP), jnp.int32),
        jax.ShapeDtypeStruct((1, 1), jnp.float32),
        jax.ShapeDtypeStruct((1, 1), jnp.float32),
        jax.ShapeDtypeStruct((K, DP), jnp.float32),
    ],
    scratch_shapes=[
        pltpu.VMEM((1, K), jnp.float32),
        pltpu.VMEM((1, K), jnp.float32),
        pltpu.VMEM((1, 1), jnp.float32),
        pltpu.VMEM((1, K), jnp.float32),
    ],
)


def _sc_gather_body(table_hbm, idx_hbm, out_hbm, idx_v, rows_v, sem):
    wid = lax.axis_index("s") * 2 + lax.axis_index("c")
    pltpu.sync_copy(idx_hbm.at[wid], idx_v)
    copies = [
        pltpu.async_copy(table_hbm.at[idx_v.at[j]],
                         rows_v.at[pl.ds(j * DP, DP)], sem)
        for j in range(NCH)
    ]
    for c in copies:
        c.wait()
    pltpu.sync_copy(rows_v, out_hbm.at[pl.ds(wid * B_PER_W, B_PER_W)])


@functools.cache
def _sc_gather():
    return functools.partial(
        pl.kernel,
        mesh=plsc.VectorSubcoreMesh(core_axis_name="c", subcore_axis_name="s"),
        out_type=jax.ShapeDtypeStruct((N, DP), jnp.float32),
        scratch_types=[
            pltpu.VMEM((NCH, DP), jnp.int32),
            pltpu.VMEM((B_PER_W, DP), jnp.float32),
            pltpu.SemaphoreType.DMA,
        ],
    )(_sc_gather_body)


def kernel(latents, prototypes):
    idx3, loss, ent, table = _tc_call(latents.T, prototypes.T)
    quantized = _sc_gather()(table, idx3)[:, :D]
    return quantized, loss[0, 0], ent[0, 0]
